# baseline (speedup 1.0000x reference)
import functools

import jax
import jax.numpy as jnp
from jax import lax
from jax.experimental import pallas as pl
from jax.experimental.pallas import tpu as pltpu
from jax.experimental.pallas import tpu_sc as plsc

B = 16384
F = 26
V = 100000
D = 16
N = 13

_NW = 32
_ROWS = B * F
_RPW = _ROWS // _NW
_CHUNK = 1024
_NCHUNK = _RPW // _CHUNK


def _sc_gather(table_hbm, idx_hbm, out_hbm, idx_v, rows_v, sem):
    wid = lax.axis_index("s") * 2 + lax.axis_index("c")
    base = wid * _RPW
    pltpu.sync_copy(idx_hbm.at[pl.ds(base, _RPW)], idx_v)
    for it in range(_NCHUNK):
        off = it * _CHUNK
        pltpu.async_copy(
            table_hbm.at[idx_v.at[pl.ds(off, _CHUNK)]], rows_v, sem
        ).wait()
        pltpu.sync_copy(rows_v, out_hbm.at[pl.ds(base + off, _CHUNK)])


@functools.partial(jax.jit, static_argnums=())
def _gather_rows(tables_flat, idx_flat):
    mesh = plsc.VectorSubcoreMesh(core_axis_name="c", subcore_axis_name="s")
    kern = functools.partial(
        pl.kernel,
        mesh=mesh,
        out_type=jax.ShapeDtypeStruct((_ROWS, D), jnp.float32),
        scratch_types=[
            pltpu.VMEM((_RPW,), jnp.int32),
            pltpu.VMEM((_CHUNK, D), jnp.float32),
            pltpu.SemaphoreType.DMA,
        ],
        compiler_params=pltpu.CompilerParams(use_tc_tiling_on_sc=False),
    )(_sc_gather)
    return kern(tables_flat, idx_flat)


_BLK = 1024
_NB = B // _BLK


def _tc_body(num_ref, emb_ref, gn_ref, ge_ref, bn_ref, be_ref, wn_ref,
             we_ref, bias_ref, out_ref, sn_ref, se_ref, qn_ref, qe_ref):
    p = pl.program_id(0)
    i = pl.program_id(1)
    nb = num_ref[...]
    eb = emb_ref[...]

    @pl.when(p == 0)
    def _phase0():
        s_n = jnp.sum(nb, axis=0, keepdims=True)
        s_e = jnp.sum(eb, axis=0, keepdims=True)
        q_n = jnp.sum(nb * nb, axis=0, keepdims=True)
        q_e = jnp.sum(eb * eb, axis=0, keepdims=True)

        @pl.when(i == 0)
        def _init():
            sn_ref[...] = s_n
            se_ref[...] = s_e
            qn_ref[...] = q_n
            qe_ref[...] = q_e

        @pl.when(i != 0)
        def _acc():
            sn_ref[...] += s_n
            se_ref[...] += s_e
            qn_ref[...] += q_n
            qe_ref[...] += q_e

    @pl.when(p == 1)
    def _phase1():
        inv_b = 1.0 / float(B)
        mn = sn_ref[...] * inv_b
        me = se_ref[...] * inv_b
        vn = qn_ref[...] * inv_b - mn * mn
        ve = qe_ref[...] * inv_b - me * me
        rn = lax.rsqrt(vn + 1e-3)
        re = lax.rsqrt(ve + 1e-3)
        an = wn_ref[...] * gn_ref[...] * rn
        ae = we_ref[...] * ge_ref[...] * re
        c = (jnp.sum(wn_ref[...] * (bn_ref[...] - gn_ref[...] * mn * rn))
             + jnp.sum(we_ref[...] * (be_ref[...] - ge_ref[...] * me * re))
             + bias_ref[0, 0])
        logit = (jnp.sum(nb * an, axis=1, keepdims=True)
                 + jnp.sum(eb * ae, axis=1, keepdims=True) + c)
        out_ref[...] = jax.nn.sigmoid(logit)


def _tc_head(numeric, emb, gn, ge, bn, be, wn, we, bias):
    vec_n = pl.BlockSpec((1, N), lambda p, i: (0, 0))
    vec_e = pl.BlockSpec((1, F * D), lambda p, i: (0, 0))
    return pl.pallas_call(
        _tc_body,
        grid=(2, _NB),
        in_specs=[
            pl.BlockSpec((_BLK, N), lambda p, i: (i, 0)),
            pl.BlockSpec((_BLK, F * D), lambda p, i: (i, 0)),
            vec_n, vec_e, vec_n, vec_e, vec_n, vec_e,
            pl.BlockSpec((1, 1), lambda p, i: (0, 0)),
        ],
        out_specs=pl.BlockSpec((_BLK, 1), lambda p, i: (i, 0)),
        out_shape=jax.ShapeDtypeStruct((B, 1), jnp.float32),
        scratch_shapes=[
            pltpu.VMEM((1, N), jnp.float32),
            pltpu.VMEM((1, F * D), jnp.float32),
            pltpu.VMEM((1, N), jnp.float32),
            pltpu.VMEM((1, F * D), jnp.float32),
        ],
    )(numeric, emb, gn, ge, bn, be, wn, we, bias)


def kernel(indices, numeric, tables, gamma, beta, W, b):
    idx_flat = (indices.astype(jnp.int32)
                + (jnp.arange(F, dtype=jnp.int32) * V)[None, :]).reshape(-1)
    tables_flat = tables.reshape(F * V, D)
    emb = _gather_rows(tables_flat, idx_flat).reshape(B, F * D)

    gn, ge = gamma[:N].reshape(1, N), gamma[N:].reshape(1, F * D)
    bn, be = beta[:N].reshape(1, N), beta[N:].reshape(1, F * D)
    w = W.reshape(-1)
    wn, we = w[:N].reshape(1, N), w[N:].reshape(1, F * D)
    bias = b.reshape(1, 1)
    return _tc_head(numeric, emb, gn, ge, bn, be, wn, we, bias)


# speedup vs baseline: 4.6678x; 4.6678x over previous
import functools

import jax
import jax.numpy as jnp
from jax import lax
from jax.experimental import pallas as pl
from jax.experimental.pallas import tpu as pltpu
from jax.experimental.pallas import tpu_sc as plsc

B = 16384
F = 26
V = 100000
D = 16
N = 13

_NW = 32
_ROWS = F * D
_RPW = _ROWS // _NW
_HALF = B // 2


def _sc_gather(tbl_hbm, idx_hbm, out_hbm, row_v, idx_v, out_v):
    wid = lax.axis_index("s") * 2 + lax.axis_index("c")
    t0 = wid * _RPW

    def load_idx(f):
        pltpu.sync_copy(idx_hbm.at[f], idx_v)

    def gather_half(h):
        def body(i, carry):
            idx16 = idx_v[pl.ds(h * _HALF + i * 16, 16)]
            out_v[pl.ds(i * 16, 16)] = plsc.load_gather(row_v, [idx16])
            return carry
        lax.fori_loop(0, _HALF // 16, body, 0, unroll=4)

    for j in range(_RPW):
        t = t0 + j
        f = t // D
        d = t % D
        if j == 0:
            load_idx(f)
        else:
            @pl.when(d == 0)
            def _reload():
                load_idx(f)
        pltpu.sync_copy(tbl_hbm.at[f, d], row_v)
        for h in range(2):
            gather_half(h)
            pltpu.sync_copy(out_v, out_hbm.at[t, pl.ds(h * _HALF, _HALF)])


def _gather_embT(tbl_T, idx_T):
    mesh = plsc.VectorSubcoreMesh(core_axis_name="c", subcore_axis_name="s")
    kern = functools.partial(
        pl.kernel,
        mesh=mesh,
        out_type=jax.ShapeDtypeStruct((_ROWS, B), jnp.float32),
        scratch_types=[
            pltpu.VMEM((V,), jnp.float32),
            pltpu.VMEM((B,), jnp.int32),
            pltpu.VMEM((_HALF,), jnp.float32),
        ],
        compiler_params=pltpu.CompilerParams(
            use_tc_tiling_on_sc=True, needs_layout_passes=False),
    )(_sc_gather)
    return kern(tbl_T, idx_T)


_BLK = 1024
_NB = B // _BLK


def _tc_body(num_ref, emb_ref, gn_ref, ge_ref, bn_ref, be_ref, wn_ref,
             we_ref, bias_ref, out_ref, sn_ref, se_ref, qn_ref, qe_ref):
    p = pl.program_id(0)
    i = pl.program_id(1)
    nb = num_ref[...]
    eb = emb_ref[...]

    @pl.when(p == 0)
    def _phase0():
        s_n = jnp.sum(nb, axis=1, keepdims=True)
        s_e = jnp.sum(eb, axis=1, keepdims=True)
        q_n = jnp.sum(nb * nb, axis=1, keepdims=True)
        q_e = jnp.sum(eb * eb, axis=1, keepdims=True)

        @pl.when(i == 0)
        def _init():
            sn_ref[...] = s_n
            se_ref[...] = s_e
            qn_ref[...] = q_n
            qe_ref[...] = q_e

        @pl.when(i != 0)
        def _acc():
            sn_ref[...] += s_n
            se_ref[...] += s_e
            qn_ref[...] += q_n
            qe_ref[...] += q_e

    @pl.when(p == 1)
    def _phase1():
        inv_b = 1.0 / float(B)
        mn = sn_ref[...] * inv_b
        me = se_ref[...] * inv_b
        vn = qn_ref[...] * inv_b - mn * mn
        ve = qe_ref[...] * inv_b - me * me
        rn = lax.rsqrt(vn + 1e-3)
        re = lax.rsqrt(ve + 1e-3)
        an = wn_ref[...] * gn_ref[...] * rn
        ae = we_ref[...] * ge_ref[...] * re
        c = (jnp.sum(wn_ref[...] * (bn_ref[...] - gn_ref[...] * mn * rn))
             + jnp.sum(we_ref[...] * (be_ref[...] - ge_ref[...] * me * re))
             + bias_ref[0, 0])
        logit = (jnp.sum(nb * an, axis=0, keepdims=True)
                 + jnp.sum(eb * ae, axis=0, keepdims=True) + c)
        out_ref[...] = jax.nn.sigmoid(logit)


def _tc_head(numeric_T, emb_T, gn, ge, bn, be, wn, we, bias):
    vec_n = pl.BlockSpec((N, 1), lambda p, i: (0, 0))
    vec_e = pl.BlockSpec((F * D, 1), lambda p, i: (0, 0))
    return pl.pallas_call(
        _tc_body,
        grid=(2, _NB),
        in_specs=[
            pl.BlockSpec((N, _BLK), lambda p, i: (0, i)),
            pl.BlockSpec((F * D, _BLK), lambda p, i: (0, i)),
            vec_n, vec_e, vec_n, vec_e, vec_n, vec_e,
            pl.BlockSpec((1, 1), lambda p, i: (0, 0)),
        ],
        out_specs=pl.BlockSpec((1, _BLK), lambda p, i: (0, i)),
        out_shape=jax.ShapeDtypeStruct((1, B), jnp.float32),
        scratch_shapes=[
            pltpu.VMEM((N, 1), jnp.float32),
            pltpu.VMEM((F * D, 1), jnp.float32),
            pltpu.VMEM((N, 1), jnp.float32),
            pltpu.VMEM((F * D, 1), jnp.float32),
        ],
    )(numeric_T, emb_T, gn, ge, bn, be, wn, we, bias)


def kernel(indices, numeric, tables, gamma, beta, W, b):
    tbl_T = jnp.transpose(tables, (0, 2, 1))
    idx_T = jnp.transpose(indices.astype(jnp.int32))
    num_T = jnp.transpose(numeric)
    emb_T = _gather_embT(tbl_T, idx_T)

    gn, ge = gamma[:N].reshape(N, 1), gamma[N:].reshape(F * D, 1)
    bn, be = beta[:N].reshape(N, 1), beta[N:].reshape(F * D, 1)
    w = W.reshape(-1)
    wn, we = w[:N].reshape(N, 1), w[N:].reshape(F * D, 1)
    bias = b.reshape(1, 1)
    out = _tc_head(num_T, emb_T, gn, ge, bn, be, wn, we, bias)
    return out.reshape(B, 1)


# speedup vs baseline: 7.8159x; 1.6744x over previous
import functools

import jax
import jax.numpy as jnp
from jax import lax
from jax.experimental import pallas as pl
from jax.experimental.pallas import tpu as pltpu
from jax.experimental.pallas import tpu_sc as plsc

B = 16384
F = 26
V = 100000
D = 16
N = 13

_NW = 32
_ROWS = F * D
_RPW = _ROWS // _NW
_HALF = B // 2


def _sc_gather(tbl_hbm, idx_hbm, out_hbm, row_v, idx_v, out_v):
    wid = lax.axis_index("s") * 2 + lax.axis_index("c")
    t0 = wid * _RPW

    def load_idx(f):
        pltpu.sync_copy(idx_hbm.at[f], idx_v)

    def gather_half(h):
        @plsc.parallel_loop(0, _HALF // 16, unroll=8)
        def _body(i):
            idx16 = idx_v[pl.ds(h * _HALF + i * 16, 16)]
            out_v[pl.ds(i * 16, 16)] = plsc.load_gather(row_v, [idx16])

    for j in range(_RPW):
        t = t0 + j
        f = t // D
        d = t % D
        if j == 0:
            load_idx(f)
        else:
            @pl.when(d == 0)
            def _reload():
                load_idx(f)
        pltpu.sync_copy(tbl_hbm.at[f, d], row_v)
        for h in range(2):
            gather_half(h)
            pltpu.sync_copy(out_v, out_hbm.at[t, pl.ds(h * _HALF, _HALF)])


def _gather_embT(tbl_T, idx_T):
    mesh = plsc.VectorSubcoreMesh(core_axis_name="c", subcore_axis_name="s")
    kern = functools.partial(
        pl.kernel,
        mesh=mesh,
        out_type=jax.ShapeDtypeStruct((_ROWS, B), jnp.float32),
        scratch_types=[
            pltpu.VMEM((V,), jnp.float32),
            pltpu.VMEM((B,), jnp.int32),
            pltpu.VMEM((_HALF,), jnp.float32),
        ],
        compiler_params=pltpu.CompilerParams(
            use_tc_tiling_on_sc=True, needs_layout_passes=False),
    )(_sc_gather)
    return kern(tbl_T, idx_T)


_BLK = 1024
_NB = B // _BLK


def _tc_body(num_ref, emb_ref, gn_ref, ge_ref, bn_ref, be_ref, wn_ref,
             we_ref, bias_ref, out_ref, sn_ref, se_ref, qn_ref, qe_ref):
    p = pl.program_id(0)
    i = pl.program_id(1)
    nb = num_ref[...]
    eb = emb_ref[...]

    @pl.when(p == 0)
    def _phase0():
        s_n = jnp.sum(nb, axis=1, keepdims=True)
        s_e = jnp.sum(eb, axis=1, keepdims=True)
        q_n = jnp.sum(nb * nb, axis=1, keepdims=True)
        q_e = jnp.sum(eb * eb, axis=1, keepdims=True)

        @pl.when(i == 0)
        def _init():
            sn_ref[...] = s_n
            se_ref[...] = s_e
            qn_ref[...] = q_n
            qe_ref[...] = q_e

        @pl.when(i != 0)
        def _acc():
            sn_ref[...] += s_n
            se_ref[...] += s_e
            qn_ref[...] += q_n
            qe_ref[...] += q_e

    @pl.when(p == 1)
    def _phase1():
        inv_b = 1.0 / float(B)
        mn = sn_ref[...] * inv_b
        me = se_ref[...] * inv_b
        vn = qn_ref[...] * inv_b - mn * mn
        ve = qe_ref[...] * inv_b - me * me
        rn = lax.rsqrt(vn + 1e-3)
        re = lax.rsqrt(ve + 1e-3)
        an = wn_ref[...] * gn_ref[...] * rn
        ae = we_ref[...] * ge_ref[...] * re
        c = (jnp.sum(wn_ref[...] * (bn_ref[...] - gn_ref[...] * mn * rn))
             + jnp.sum(we_ref[...] * (be_ref[...] - ge_ref[...] * me * re))
             + bias_ref[0, 0])
        logit = (jnp.sum(nb * an, axis=0, keepdims=True)
                 + jnp.sum(eb * ae, axis=0, keepdims=True) + c)
        out_ref[...] = jax.nn.sigmoid(logit)


def _tc_head(numeric_T, emb_T, gn, ge, bn, be, wn, we, bias):
    vec_n = pl.BlockSpec((N, 1), lambda p, i: (0, 0))
    vec_e = pl.BlockSpec((F * D, 1), lambda p, i: (0, 0))
    return pl.pallas_call(
        _tc_body,
        grid=(2, _NB),
        in_specs=[
            pl.BlockSpec((N, _BLK), lambda p, i: (0, i)),
            pl.BlockSpec((F * D, _BLK), lambda p, i: (0, i)),
            vec_n, vec_e, vec_n, vec_e, vec_n, vec_e,
            pl.BlockSpec((1, 1), lambda p, i: (0, 0)),
        ],
        out_specs=pl.BlockSpec((1, _BLK), lambda p, i: (0, i)),
        out_shape=jax.ShapeDtypeStruct((1, B), jnp.float32),
        scratch_shapes=[
            pltpu.VMEM((N, 1), jnp.float32),
            pltpu.VMEM((F * D, 1), jnp.float32),
            pltpu.VMEM((N, 1), jnp.float32),
            pltpu.VMEM((F * D, 1), jnp.float32),
        ],
    )(numeric_T, emb_T, gn, ge, bn, be, wn, we, bias)


def kernel(indices, numeric, tables, gamma, beta, W, b):
    tbl_T = jnp.transpose(tables, (0, 2, 1))
    idx_T = jnp.transpose(indices.astype(jnp.int32))
    num_T = jnp.transpose(numeric)
    emb_T = _gather_embT(tbl_T, idx_T)

    gn, ge = gamma[:N].reshape(N, 1), gamma[N:].reshape(F * D, 1)
    bn, be = beta[:N].reshape(N, 1), beta[N:].reshape(F * D, 1)
    w = W.reshape(-1)
    wn, we = w[:N].reshape(N, 1), w[N:].reshape(F * D, 1)
    bias = b.reshape(1, 1)
    out = _tc_head(num_T, emb_T, gn, ge, bn, be, wn, we, bias)
    return out.reshape(B, 1)


# speedup vs baseline: 8.1202x; 1.0389x over previous
import functools

import jax
import jax.numpy as jnp
from jax import lax
from jax.experimental import pallas as pl
from jax.experimental.pallas import tpu as pltpu
from jax.experimental.pallas import tpu_sc as plsc

B = 16384
F = 26
V = 100000
D = 16
N = 13

_NW = 32
_ROWS = F * D
_RPW = _ROWS // _NW
_HALF = B // 2


def _sc_gather(tbl_hbm, idx_hbm, out_hbm, row_v, idx_v, out_v):
    wid = lax.axis_index("s") * 2 + lax.axis_index("c")
    t0 = wid * _RPW

    def load_idx(f):
        pltpu.sync_copy(idx_hbm.at[f], idx_v)

    def gather_half(h):
        @plsc.parallel_loop(0, _HALF // 16, unroll=8)
        def _body(i):
            idx16 = idx_v[pl.ds(h * _HALF + i * 16, 16)]
            out_v[pl.ds(i * 16, 16)] = plsc.load_gather(row_v, [idx16])

    for j in range(_RPW):
        t = t0 + j
        f = t // D
        d = t % D
        if j == 0:
            load_idx(f)
        else:
            @pl.when(d == 0)
            def _reload():
                load_idx(f)
        pltpu.sync_copy(tbl_hbm.at[f, d], row_v)
        for h in range(2):
            gather_half(h)
            pltpu.sync_copy(out_v, out_hbm.at[t, pl.ds(h * _HALF, _HALF)])


def _gather_embT(tbl_T, idx_T):
    mesh = plsc.VectorSubcoreMesh(core_axis_name="c", subcore_axis_name="s")
    kern = functools.partial(
        pl.kernel,
        mesh=mesh,
        out_type=jax.ShapeDtypeStruct((_ROWS, B), jnp.float32),
        scratch_types=[
            pltpu.VMEM((V,), jnp.float32),
            pltpu.VMEM((B,), jnp.int32),
            pltpu.VMEM((_HALF,), jnp.float32),
        ],
        compiler_params=pltpu.CompilerParams(
            use_tc_tiling_on_sc=True, needs_layout_passes=False),
    )(_sc_gather)
    return kern(tbl_T, idx_T)


_BLK = 2048
_NB = B // _BLK


def _tc_body(num_ref, emb_ref, gn_ref, ge_ref, bn_ref, be_ref, wn_ref,
             we_ref, bias_ref, out_ref, sn_ref, se_ref, qn_ref, qe_ref):
    p = pl.program_id(0)
    i = pl.program_id(1)
    nb = num_ref[...]
    eb = emb_ref[...]

    ones = jnp.ones((_BLK, 1), dtype=jnp.float32)

    @pl.when(p == 0)
    def _phase0():
        s_n = jnp.sum(nb, axis=1, keepdims=True)
        s_e = jax.lax.dot(eb, ones)
        q_n = jnp.sum(nb * nb, axis=1, keepdims=True)
        q_e = jax.lax.dot(eb * eb, ones)

        @pl.when(i == 0)
        def _init():
            sn_ref[...] = s_n
            se_ref[...] = s_e
            qn_ref[...] = q_n
            qe_ref[...] = q_e

        @pl.when(i != 0)
        def _acc():
            sn_ref[...] += s_n
            se_ref[...] += s_e
            qn_ref[...] += q_n
            qe_ref[...] += q_e

    @pl.when(p == 1)
    def _phase1():
        inv_b = 1.0 / float(B)
        mn = sn_ref[...] * inv_b
        me = se_ref[...] * inv_b
        vn = qn_ref[...] * inv_b - mn * mn
        ve = qe_ref[...] * inv_b - me * me
        rn = lax.rsqrt(vn + 1e-3)
        re = lax.rsqrt(ve + 1e-3)
        an = wn_ref[...] * gn_ref[...] * rn
        ae = we_ref[...] * ge_ref[...] * re
        c = (jnp.sum(wn_ref[...] * (bn_ref[...] - gn_ref[...] * mn * rn))
             + jnp.sum(we_ref[...] * (be_ref[...] - ge_ref[...] * me * re))
             + bias_ref[0, 0])
        logit = (jnp.sum(nb * an, axis=0, keepdims=True)
                 + jax.lax.dot_general(ae, eb, (((0,), (0,)), ((), ())))
                 + c)
        out_ref[...] = jax.nn.sigmoid(logit)


def _tc_head(numeric_T, emb_T, gn, ge, bn, be, wn, we, bias):
    vec_n = pl.BlockSpec((N, 1), lambda p, i: (0, 0))
    vec_e = pl.BlockSpec((F * D, 1), lambda p, i: (0, 0))
    return pl.pallas_call(
        _tc_body,
        grid=(2, _NB),
        in_specs=[
            pl.BlockSpec((N, _BLK), lambda p, i: (0, i)),
            pl.BlockSpec((F * D, _BLK), lambda p, i: (0, i)),
            vec_n, vec_e, vec_n, vec_e, vec_n, vec_e,
            pl.BlockSpec((1, 1), lambda p, i: (0, 0)),
        ],
        out_specs=pl.BlockSpec((1, _BLK), lambda p, i: (0, i)),
        out_shape=jax.ShapeDtypeStruct((1, B), jnp.float32),
        scratch_shapes=[
            pltpu.VMEM((N, 1), jnp.float32),
            pltpu.VMEM((F * D, 1), jnp.float32),
            pltpu.VMEM((N, 1), jnp.float32),
            pltpu.VMEM((F * D, 1), jnp.float32),
        ],
    )(numeric_T, emb_T, gn, ge, bn, be, wn, we, bias)


def kernel(indices, numeric, tables, gamma, beta, W, b):
    tbl_T = jnp.transpose(tables, (0, 2, 1))
    idx_T = jnp.transpose(indices.astype(jnp.int32))
    num_T = jnp.transpose(numeric)
    emb_T = _gather_embT(tbl_T, idx_T)

    gn, ge = gamma[:N].reshape(N, 1), gamma[N:].reshape(F * D, 1)
    bn, be = beta[:N].reshape(N, 1), beta[N:].reshape(F * D, 1)
    w = W.reshape(-1)
    wn, we = w[:N].reshape(N, 1), w[N:].reshape(F * D, 1)
    bias = b.reshape(1, 1)
    out = _tc_head(num_T, emb_T, gn, ge, bn, be, wn, we, bias)
    return out.reshape(B, 1)


# speedup vs baseline: 8.2000x; 1.0098x over previous
import functools

import jax
import jax.numpy as jnp
from jax import lax
from jax.experimental import pallas as pl
from jax.experimental.pallas import tpu as pltpu
from jax.experimental.pallas import tpu_sc as plsc

B = 16384
F = 26
V = 100000
D = 16
N = 13

_NW = 32
_ROWS = F * D
_RPW = _ROWS // _NW
_HALF = B // 2


_VH = 50048


def _sc_gather(tbl_hbm, idx_hbm, out_hbm, row_v, idx_v, out_v,
               sem_row, sem_idx, sem_out):
    wid = lax.axis_index("s") * 2 + lax.axis_index("c")
    t0 = wid * _RPW

    def load_idx(f):
        pltpu.async_copy(idx_hbm.at[f], idx_v, sem_idx).wait()

    def gather_half(h):
        @plsc.parallel_loop(0, _HALF // 16, unroll=8)
        def _body(i):
            idx16 = idx_v[pl.ds(h * _HALF + i * 16, 16)]
            out_v[pl.ds(i * 16, 16)] = plsc.load_gather(row_v, [idx16])

    out_dma = None
    for j in range(_RPW):
        t = t0 + j
        f = t // D
        d = t % D
        c0 = pltpu.async_copy(tbl_hbm.at[f, d], row_v, sem_row)
        if j == 0:
            load_idx(f)
        else:
            @pl.when(d == 0)
            def _reload():
                load_idx(f)
        c0.wait()
        for h in range(2):
            if out_dma is not None:
                out_dma.wait()
            gather_half(h)
            out_dma = pltpu.async_copy(
                out_v, out_hbm.at[t, pl.ds(h * _HALF, _HALF)], sem_out)
    out_dma.wait()


def _gather_embT(tbl_T, idx_T):
    mesh = plsc.VectorSubcoreMesh(core_axis_name="c", subcore_axis_name="s")
    kern = functools.partial(
        pl.kernel,
        mesh=mesh,
        out_type=jax.ShapeDtypeStruct((_ROWS, B), jnp.float32),
        scratch_types=[
            pltpu.VMEM((V,), jnp.float32),
            pltpu.VMEM((B,), jnp.int32),
            pltpu.VMEM((_HALF,), jnp.float32),
            pltpu.SemaphoreType.DMA,
            pltpu.SemaphoreType.DMA,
            pltpu.SemaphoreType.DMA,
        ],
        compiler_params=pltpu.CompilerParams(
            use_tc_tiling_on_sc=True, needs_layout_passes=False),
    )(_sc_gather)
    return kern(tbl_T, idx_T)


_BLK = 2048
_NB = B // _BLK


def _tc_body(num_ref, emb_ref, gn_ref, ge_ref, bn_ref, be_ref, wn_ref,
             we_ref, bias_ref, out_ref, sn_ref, se_ref, qn_ref, qe_ref):
    p = pl.program_id(0)
    i = pl.program_id(1)
    nb = num_ref[...]
    eb = emb_ref[...]

    ones = jnp.ones((_BLK, 1), dtype=jnp.float32)

    @pl.when(p == 0)
    def _phase0():
        s_n = jnp.sum(nb, axis=1, keepdims=True)
        s_e = jax.lax.dot(eb, ones)
        q_n = jnp.sum(nb * nb, axis=1, keepdims=True)
        q_e = jax.lax.dot(eb * eb, ones)

        @pl.when(i == 0)
        def _init():
            sn_ref[...] = s_n
            se_ref[...] = s_e
            qn_ref[...] = q_n
            qe_ref[...] = q_e

        @pl.when(i != 0)
        def _acc():
            sn_ref[...] += s_n
            se_ref[...] += s_e
            qn_ref[...] += q_n
            qe_ref[...] += q_e

    @pl.when(p == 1)
    def _phase1():
        inv_b = 1.0 / float(B)
        mn = sn_ref[...] * inv_b
        me = se_ref[...] * inv_b
        vn = qn_ref[...] * inv_b - mn * mn
        ve = qe_ref[...] * inv_b - me * me
        rn = lax.rsqrt(vn + 1e-3)
        re = lax.rsqrt(ve + 1e-3)
        an = wn_ref[...] * gn_ref[...] * rn
        ae = we_ref[...] * ge_ref[...] * re
        c = (jnp.sum(wn_ref[...] * (bn_ref[...] - gn_ref[...] * mn * rn))
             + jnp.sum(we_ref[...] * (be_ref[...] - ge_ref[...] * me * re))
             + bias_ref[0, 0])
        logit = (jnp.sum(nb * an, axis=0, keepdims=True)
                 + jax.lax.dot_general(ae, eb, (((0,), (0,)), ((), ())))
                 + c)
        out_ref[...] = jax.nn.sigmoid(logit)


def _tc_head(numeric_T, emb_T, gn, ge, bn, be, wn, we, bias):
    vec_n = pl.BlockSpec((N, 1), lambda p, i: (0, 0))
    vec_e = pl.BlockSpec((F * D, 1), lambda p, i: (0, 0))
    return pl.pallas_call(
        _tc_body,
        grid=(2, _NB),
        in_specs=[
            pl.BlockSpec((N, _BLK), lambda p, i: (0, i)),
            pl.BlockSpec((F * D, _BLK), lambda p, i: (0, i)),
            vec_n, vec_e, vec_n, vec_e, vec_n, vec_e,
            pl.BlockSpec((1, 1), lambda p, i: (0, 0)),
        ],
        out_specs=pl.BlockSpec((1, _BLK), lambda p, i: (0, i)),
        out_shape=jax.ShapeDtypeStruct((1, B), jnp.float32),
        scratch_shapes=[
            pltpu.VMEM((N, 1), jnp.float32),
            pltpu.VMEM((F * D, 1), jnp.float32),
            pltpu.VMEM((N, 1), jnp.float32),
            pltpu.VMEM((F * D, 1), jnp.float32),
        ],
    )(numeric_T, emb_T, gn, ge, bn, be, wn, we, bias)


def kernel(indices, numeric, tables, gamma, beta, W, b):
    tbl_T = jnp.transpose(tables, (0, 2, 1))
    idx_T = jnp.transpose(indices.astype(jnp.int32))
    num_T = jnp.transpose(numeric)
    emb_T = _gather_embT(tbl_T, idx_T)

    gn, ge = gamma[:N].reshape(N, 1), gamma[N:].reshape(F * D, 1)
    bn, be = beta[:N].reshape(N, 1), beta[N:].reshape(F * D, 1)
    w = W.reshape(-1)
    wn, we = w[:N].reshape(N, 1), w[N:].reshape(F * D, 1)
    bias = b.reshape(1, 1)
    out = _tc_head(num_T, emb_T, gn, ge, bn, be, wn, we, bias)
    return out.reshape(B, 1)
